# baseline (speedup 1.0000x reference)
import functools

import jax
import jax.numpy as jnp
from jax import lax
from jax.experimental import pallas as pl
from jax.experimental.pallas import tpu as pltpu
from jax.experimental.pallas import tpu_sc as plsc

NC, NS = 2, 16
NW = NC * NS
CHUNK = 128
LANES = 16


def kernel(words, tags, word_table, pos_table):
    B, L = words.shape
    D = word_table.shape[1]
    T = pos_table.shape[0]
    N = B * L
    n_per_w = N // NW
    n_chunks = n_per_w // CHUNK

    words_flat = words.reshape(N).astype(jnp.int32)
    tags_flat = tags.reshape(N).astype(jnp.int32)
    wtab128 = jnp.pad(word_table, ((0, 0), (0, D)))

    mesh = plsc.VectorSubcoreMesh(
        core_axis_name="c", subcore_axis_name="s",
        num_cores=NC, num_subcores=NS)

    @functools.partial(
        pl.kernel,
        out_type=jax.ShapeDtypeStruct((N, 2 * D), jnp.float32),
        mesh=mesh,
        compiler_params=pltpu.CompilerParams(needs_layout_passes=False),
        scratch_types=[
            pltpu.VMEM((CHUNK,), jnp.int32),
            pltpu.VMEM((CHUNK,), jnp.int32),
            pltpu.VMEM((T, D), jnp.float32),
            pltpu.VMEM((CHUNK, 2 * D), jnp.float32),
            pltpu.SemaphoreType.DMA,
        ],
    )
    def run(words_hbm, tags_hbm, wtab_hbm, ptab_hbm, out_hbm,
            widx, tidx, ptab, comb, sem_w):
        wid = lax.axis_index("s") * NC + lax.axis_index("c")
        base0 = wid * n_per_w
        pltpu.sync_copy(ptab_hbm, ptab)
        row_ids = [lax.iota(jnp.int32, LANES) + LANES * g
                   for g in range(CHUNK // LANES)]

        def body(i, carry):
            base = base0 + i * CHUNK
            pltpu.sync_copy(words_hbm.at[pl.ds(base, CHUNK)], widx)
            pltpu.sync_copy(tags_hbm.at[pl.ds(base, CHUNK)], tidx)
            pltpu.async_copy(wtab_hbm.at[widx], comb, sem_w).wait()

            def fill(j, c):
                col = jnp.full((LANES,), j, jnp.int32)
                for g in range(CHUNK // LANES):
                    tvec = tidx[pl.ds(LANES * g, LANES)]
                    vals = plsc.load_gather(ptab, [tvec, col])
                    plsc.store_scatter(comb, [row_ids[g], col + D], vals)
                return c

            lax.fori_loop(0, D, fill, 0)
            pltpu.sync_copy(comb, out_hbm.at[pl.ds(base, CHUNK), :])
            return carry

        lax.fori_loop(0, n_chunks, body, 0)

    out = run(words_flat, tags_flat, wtab128, pos_table)
    return out.reshape(B, L, 2 * D)


# speedup vs baseline: 1.9486x; 1.9486x over previous
import functools

import jax
import jax.numpy as jnp
from jax import lax
from jax.experimental import pallas as pl
from jax.experimental.pallas import tpu as pltpu
from jax.experimental.pallas import tpu_sc as plsc

NC, NS = 2, 16
NW = NC * NS
CHUNK = 128
LANES = 16


def kernel(words, tags, word_table, pos_table):
    B, L = words.shape
    D = word_table.shape[1]
    T = pos_table.shape[0]
    N = B * L
    n_per_w = N // NW
    n_chunks = n_per_w // CHUNK

    words_flat = words.reshape(N).astype(jnp.int32)
    tags_flat = tags.reshape(N).astype(jnp.int32)
    wtab128 = jnp.pad(word_table, ((0, 0), (0, D)))

    mesh = plsc.VectorSubcoreMesh(
        core_axis_name="c", subcore_axis_name="s",
        num_cores=NC, num_subcores=NS)

    @functools.partial(
        pl.kernel,
        out_type=jax.ShapeDtypeStruct((N, 2 * D), jnp.float32),
        mesh=mesh,
        compiler_params=pltpu.CompilerParams(needs_layout_passes=False),
        scratch_types=[
            pltpu.VMEM((CHUNK,), jnp.int32),
            pltpu.VMEM((CHUNK,), jnp.int32),
            pltpu.VMEM((T, D), jnp.float32),
            pltpu.VMEM((CHUNK, 2 * D), jnp.float32),
            pltpu.SemaphoreType.DMA,
        ],
    )
    def run(words_hbm, tags_hbm, wtab_hbm, ptab_hbm, out_hbm,
            widx, tidx, ptab, comb, sem_w):
        wid = lax.axis_index("s") * NC + lax.axis_index("c")
        base0 = wid * n_per_w
        pltpu.sync_copy(ptab_hbm, ptab)
        row_ids = [lax.iota(jnp.int32, LANES) + LANES * g
                   for g in range(CHUNK // LANES)]

        def body(i, carry):
            base = base0 + i * CHUNK
            pltpu.sync_copy(words_hbm.at[pl.ds(base, CHUNK)], widx)
            pltpu.sync_copy(tags_hbm.at[pl.ds(base, CHUNK)], tidx)
            pltpu.async_copy(wtab_hbm.at[widx], comb, sem_w).wait()
            tvecs = [tidx[pl.ds(LANES * g, LANES)]
                     for g in range(CHUNK // LANES)]

            @plsc.parallel_loop(0, D, unroll=4)
            def fill(j):
                col = jnp.full((LANES,), j, jnp.int32)
                col_hi = col + D
                for g in range(CHUNK // LANES):
                    vals = plsc.load_gather(ptab, [tvecs[g], col])
                    plsc.store_scatter(comb, [row_ids[g], col_hi], vals)

            pltpu.sync_copy(comb, out_hbm.at[pl.ds(base, CHUNK), :])
            return carry

        lax.fori_loop(0, n_chunks, body, 0)

    out = run(words_flat, tags_flat, wtab128, pos_table)
    return out.reshape(B, L, 2 * D)


# speedup vs baseline: 4.3304x; 2.2223x over previous
import functools

import jax
import jax.numpy as jnp
from jax import lax
from jax.experimental import pallas as pl
from jax.experimental.pallas import tpu as pltpu
from jax.experimental.pallas import tpu_sc as plsc

NC, NS = 2, 16
NW = NC * NS
CHUNK = 128
LANES = 16
KBUF = 4
LEAD = 2


def kernel(words, tags, word_table, pos_table):
    B, L = words.shape
    D = word_table.shape[1]
    T = pos_table.shape[0]
    N = B * L
    n_per_w = N // NW
    n_chunks = n_per_w // CHUNK
    n_groups = n_chunks // KBUF

    words_flat = words.reshape(N).astype(jnp.int32)
    tags_flat = tags.reshape(N).astype(jnp.int32)
    wtab128 = jnp.pad(word_table, ((0, 0), (0, D)))

    mesh = plsc.VectorSubcoreMesh(
        core_axis_name="c", subcore_axis_name="s",
        num_cores=NC, num_subcores=NS)

    @functools.partial(
        pl.kernel,
        out_type=jax.ShapeDtypeStruct((N, 2 * D), jnp.float32),
        mesh=mesh,
        compiler_params=pltpu.CompilerParams(needs_layout_passes=False),
        scratch_types=[
            pltpu.VMEM((n_per_w,), jnp.int32),
            pltpu.VMEM((n_per_w,), jnp.int32),
            pltpu.VMEM((T, D), jnp.float32),
            [pltpu.VMEM((CHUNK, 2 * D), jnp.float32)] * KBUF,
            [pltpu.SemaphoreType.DMA] * KBUF,
            [pltpu.SemaphoreType.DMA] * KBUF,
        ],
    )
    def run(words_hbm, tags_hbm, wtab_hbm, ptab_hbm, out_hbm,
            widx, tidx, ptab, combs, gsems, wsems):
        wid = lax.axis_index("s") * NC + lax.axis_index("c")
        base0 = wid * n_per_w
        pltpu.sync_copy(words_hbm.at[pl.ds(base0, n_per_w)], widx)
        pltpu.sync_copy(tags_hbm.at[pl.ds(base0, n_per_w)], tidx)
        pltpu.sync_copy(ptab_hbm, ptab)

        def issue_gather(c, b):
            pltpu.async_copy(wtab_hbm.at[widx.at[pl.ds(c * CHUNK, CHUNK)]],
                             combs[b], gsems[b])

        def wait_gather(b):
            pltpu.make_async_copy(wtab_hbm.at[widx.at[pl.ds(0, CHUNK)]],
                                  combs[b], gsems[b]).wait()

        def issue_wb(c, b):
            pltpu.async_copy(combs[b],
                             out_hbm.at[pl.ds(base0 + c * CHUNK, CHUNK), :],
                             wsems[b])

        def wait_wb(b):
            pltpu.make_async_copy(combs[b],
                                  out_hbm.at[pl.ds(base0, CHUNK), :],
                                  wsems[b]).wait()

        for c in range(LEAD):
            issue_gather(c, c % KBUF)

        def group(k, carry):
            for b in range(KBUF):
                c = k * KBUF + b
                bb = (b + LEAD) % KBUF
                if b < LEAD:
                    @pl.when(k > 0)
                    def _():
                        wait_wb(bb)

                    issue_gather(c + LEAD, bb)
                else:
                    wait_wb(bb)

                    @pl.when(k < n_groups - 1)
                    def _():
                        issue_gather(c + LEAD, bb)

                wait_gather(b)
                off = c * CHUNK

                @plsc.parallel_loop(0, CHUNK // LANES)
                def fill(g):
                    tvec = tidx[pl.ds(off + g * LANES, LANES)]
                    for rr in range(LANES):
                        t = tvec[rr]
                        r = g * LANES + rr
                        for j in range(D // LANES):
                            combs[b][r, pl.ds(D + LANES * j, LANES)] = (
                                ptab[t, pl.ds(LANES * j, LANES)])

                issue_wb(c, b)
            return carry

        lax.fori_loop(0, n_groups, group, 0)
        for i in range(LEAD):
            wait_wb((n_chunks - LEAD + i) % KBUF)

    out = run(words_flat, tags_flat, wtab128, pos_table)
    return out.reshape(B, L, 2 * D)
